# baseline (speedup 1.0000x reference)
import functools

import jax
import jax.numpy as jnp
from jax import lax
from jax.experimental import pallas as pl
from jax.experimental.pallas import tpu as pltpu
from jax.experimental.pallas import tpu_sc as plsc

_M = 1000000
_D = 64
_B = 400
_NBLK = _M // _B
_NW = 32
_L = 16


def _rsqrt16(x):
    i = plsc.bitcast(x, jnp.int32)
    i = jnp.int32(0x5F3759DF) - lax.shift_right_logical(i, 1)
    y = plsc.bitcast(i, jnp.float32)
    for _ in range(3):
        y = y * (1.5 - 0.5 * x * y * y)
    return y


def _vsum(v):
    return lax.reduce_sum_p.bind(v, axes=(0,))


def _sc_body(idx_hbm, e_hbm, h_hbm, went_hbm, bent_hbm, wdelta_hbm,
             bdelta_hbm, eout_hbm, lg_hbm, part_hbm,
             buf0, buf1, lbuf0, lbuf1,
             hbuf, wbuf, bentbuf, wdbuf, bdbuf, ibuf,
             sebuf, tvbuf, obuf, in_sem, out_sem, lg_sem):
    wid = lax.axis_index("s") * 2 + lax.axis_index("c")

    pltpu.sync_copy(idx_hbm, ibuf)
    pltpu.sync_copy(h_hbm, hbuf)
    pltpu.sync_copy(went_hbm, wbuf)
    pltpu.sync_copy(bent_hbm, bentbuf)
    pltpu.sync_copy(wdelta_hbm, wdbuf)
    pltpu.sync_copy(bdelta_hbm, bdbuf)
    idx = ibuf[0:_L][0]
    iota = lax.iota(jnp.int32, _L)

    hvec = [hbuf[g * _L:(g + 1) * _L] for g in range(4)]
    bevec = [bentbuf[g * _L:(g + 1) * _L] for g in range(4)]

    pvec = []
    for jg in range(4):
        acc = jnp.zeros((_L,), jnp.float32)
        for k in range(_D):
            wrow = plsc.load_gather(wbuf, [k * _D + jg * _L + iota])
            acc = acc + hvec[k // _L][k % _L] * wrow
        pvec.append(acc)

    hbacc = jnp.zeros((_L,), jnp.float32)
    pn = jnp.zeros((_L,), jnp.float32)
    for g in range(4):
        hbacc = hbacc + hvec[g] * bevec[g]
        pn = pn + pvec[g] * pvec[g]
    hb = _vsum(hbacc)
    pnorm_s = _vsum(pn)
    shift = pnorm_s * _rsqrt16(jnp.full((_L,), pnorm_s, jnp.float32))[0] + hb

    g_owner = idx // _B
    owner_w = lax.rem(g_owner, _NW)
    owner_j = g_owner // _NW

    sebuf[...] = jnp.zeros((_L,), jnp.float32)
    tvbuf[...] = jnp.zeros((_L,), jnp.float32)

    n_j = (_NBLK - wid + _NW - 1) // _NW

    bufs = [buf0, buf1]
    lbufs = [lbuf0, lbuf1]

    def blk_start(j):
        return (wid + j * _NW) * _B

    def start_in(j, b):
        pltpu.async_copy(e_hbm.at[pl.ds(blk_start(j), _B), :], bufs[b],
                         in_sem)

    def wait_in(b):
        pltpu.make_async_copy(e_hbm.at[pl.ds(0, _B), :], bufs[b],
                              in_sem).wait()

    def start_out(j, b):
        pltpu.async_copy(bufs[b], eout_hbm.at[pl.ds(blk_start(j), _B), :],
                         out_sem)

    def wait_out(b):
        pltpu.make_async_copy(bufs[b], eout_hbm.at[pl.ds(0, _B), :],
                              out_sem).wait()

    def start_lg(j, b):
        pltpu.async_copy(lbufs[b], lg_hbm.at[pl.ds(blk_start(j), _B)], lg_sem)

    def wait_lg(b):
        pltpu.make_async_copy(lbufs[b], lg_hbm.at[pl.ds(0, _B)], lg_sem).wait()

    def compute(j, b):
        buf = bufs[b]
        lbuf = lbufs[b]

        def group_body(t, c2):
            rows = t * _L + iota
            acc = jnp.full((_L,), hb, jnp.float32)
            for d in range(_D):
                v = plsc.load_gather(buf, [rows, jnp.full((_L,), d, jnp.int32)])
                acc = acc + v * pvec[d // _L][d % _L]
            plsc.store_scatter(lbuf, [rows], acc)
            sebuf[...] = sebuf[...] + jnp.exp(acc - shift)
            return c2

        lax.fori_loop(0, _B // _L, group_body, 0)

        @pl.when((wid == owner_w) & (j == owner_j))
        def _update():
            g_own = wid + j * _NW
            lrow = idx - g_own * _B
            tg = lrow // _L
            lr = lrow - tg * _L
            lv = lbuf[pl.ds(tg * _L, _L)]
            tvbuf[...] = jnp.where(iota == lr, lv, 0.0)
            rowsplat = jnp.full((_L,), lrow, jnp.int32)
            e_g = [plsc.load_gather(buf, [rowsplat, q * _L + iota])
                   for q in range(4)]
            s = jnp.zeros((_L,), jnp.float32)
            for q in range(4):
                qacc = jnp.zeros((_L,), jnp.float32)
                for k in range(_D):
                    wcol = plsc.load_gather(
                        wdbuf, [(q * _L + iota) * _D + k])
                    qacc = qacc + e_g[k // _L][k % _L] * wcol
                qv = qacc + bdbuf[q * _L:(q + 1) * _L]
                s = s + hvec[q] * qv
            sv = jnp.full((_L,), _vsum(s), jnp.float32)
            delta = 1.0 / (1.0 + jnp.exp(-sv))
            nrm = jnp.zeros((_L,), jnp.float32)
            us = []
            for q in range(4):
                u = delta * e_g[q] + (1.0 - delta) * hvec[q]
                us.append(u)
                nrm = nrm + u * u
            rin = _rsqrt16(jnp.full((_L,), _vsum(nrm), jnp.float32))
            for q in range(4):
                plsc.store_scatter(buf, [rowsplat, q * _L + iota], us[q] * rin)

    for b in range(2):
        @pl.when(b < n_j)
        def _(b=b):
            start_in(b, b)

    n_outer = (_NBLK // _NW + 1 + 1) // 2 + 1

    def outer_body(t, carry):
        j0 = t * 2
        for b in range(2):
            @pl.when(j0 + b < n_j)
            def _(b=b):
                j = j0 + b
                wait_in(b)
                compute(j, b)
                start_out(j, b)
                start_lg(j, b)
        for b in range(2):
            @pl.when(j0 + 2 + b < n_j)
            def _(b=b):
                wait_out(b)
                wait_lg(b)
                start_in(j0 + 2 + b, b)
        return carry

    lax.fori_loop(0, n_outer, outer_body, 0)

    def drain_body(t, carry):
        wait_out(0)
        wait_lg(0)
        return carry

    lax.fori_loop(0, jnp.minimum(n_j, 2), drain_body, 0)

    se_tot = _vsum(sebuf[...])
    tv = _vsum(tvbuf[...])
    out16 = jnp.where(iota == 0, jnp.full((_L,), se_tot, jnp.float32),
                      jnp.where(iota == 1, jnp.full((_L,), tv, jnp.float32),
                                jnp.where(iota == 2,
                                          jnp.full((_L,), shift, jnp.float32),
                                          jnp.zeros((_L,), jnp.float32))))
    obuf[0:_L] = out16
    for g in range(1, 8):
        obuf[g * _L:(g + 1) * _L] = jnp.zeros((_L,), jnp.float32)
    pltpu.sync_copy(obuf, part_hbm.at[wid])


def _loss_body(part_ref, loss_ref):
    p = part_ref[...]
    lanei = jax.lax.broadcasted_iota(jnp.int32, (_NW, 128), 1)
    se = jnp.sum(jnp.where(lanei == 0, p, 0.0))
    tv = jnp.sum(jnp.where(lanei == 1, p, 0.0))
    shift = jnp.sum(jnp.where(lanei == 2, p, 0.0)) / _NW
    loss_ref[...] = jnp.full((1, 128), jnp.log(se) + shift - tv, jnp.float32)


def kernel(h, r, entity_idx, entity_embeddings, W_ent, b_ent, W_delta, b_delta):
    del r
    idx16 = jnp.broadcast_to(jnp.asarray(entity_idx, jnp.int32), (_L,))
    mesh = plsc.VectorSubcoreMesh(core_axis_name="c", subcore_axis_name="s")

    sc = functools.partial(
        pl.kernel,
        mesh=mesh,
        compiler_params=pltpu.CompilerParams(needs_layout_passes=False),
        out_type=[
            jax.ShapeDtypeStruct((_M, _D), jnp.float32),
            jax.ShapeDtypeStruct((_M,), jnp.float32),
            jax.ShapeDtypeStruct((_NW, 128), jnp.float32),
        ],
        scratch_types=[
            pltpu.VMEM((_B, _D), jnp.float32),
            pltpu.VMEM((_B, _D), jnp.float32),
            pltpu.VMEM((_B,), jnp.float32),
            pltpu.VMEM((_B,), jnp.float32),
            pltpu.VMEM((_D,), jnp.float32),
            pltpu.VMEM((_D * _D,), jnp.float32),
            pltpu.VMEM((_D,), jnp.float32),
            pltpu.VMEM((_D * _D,), jnp.float32),
            pltpu.VMEM((_D,), jnp.float32),
            pltpu.VMEM((_L,), jnp.int32),
            pltpu.VMEM((_L,), jnp.float32),
            pltpu.VMEM((_L,), jnp.float32),
            pltpu.VMEM((128,), jnp.float32),
            pltpu.SemaphoreType.DMA,
            pltpu.SemaphoreType.DMA,
            pltpu.SemaphoreType.DMA,
        ],
    )(_sc_body)

    eout, lg, part = sc(idx16, entity_embeddings, h,
                        W_ent.reshape(_D * _D), b_ent,
                        W_delta.reshape(_D * _D), b_delta)

    loss_v = pl.pallas_call(
        _loss_body,
        in_specs=[pl.BlockSpec((_NW, 128), lambda: (0, 0))],
        out_specs=pl.BlockSpec((1, 128), lambda: (0, 0)),
        out_shape=jax.ShapeDtypeStruct((1, 128), jnp.float32),
    )(part)

    return lg, loss_v[0, 0], eout


# speedup vs baseline: 1.0198x; 1.0198x over previous
import functools

import jax
import jax.numpy as jnp
from jax import lax
from jax.experimental import pallas as pl
from jax.experimental.pallas import tpu as pltpu
from jax.experimental.pallas import tpu_sc as plsc

_M = 1000000
_D = 64
_B = 160
_NBLK = _M // _B
_NW = 32
_L = 16


def _rsqrt16(x):
    i = plsc.bitcast(x, jnp.int32)
    i = jnp.int32(0x5F3759DF) - lax.shift_right_logical(i, 1)
    y = plsc.bitcast(i, jnp.float32)
    for _ in range(3):
        y = y * (1.5 - 0.5 * x * y * y)
    return y


def _vsum(v):
    return lax.reduce_sum_p.bind(v, axes=(0,))


def _sc_body(idx_hbm, e_hbm, h_hbm, went_hbm, bent_hbm, wdelta_hbm,
             bdelta_hbm, eout_hbm, lg_hbm, part_hbm,
             buf0, buf1, buf2, buf3, lbuf0, lbuf1, lbuf2, lbuf3,
             hbuf, wbuf, bentbuf, wdbuf, bdbuf, ibuf,
             sebuf, tvbuf, obuf, in_sem, out_sem, lg_sem):
    wid = lax.axis_index("s") * 2 + lax.axis_index("c")

    pltpu.sync_copy(idx_hbm, ibuf)
    pltpu.sync_copy(h_hbm, hbuf)
    pltpu.sync_copy(went_hbm, wbuf)
    pltpu.sync_copy(bent_hbm, bentbuf)
    pltpu.sync_copy(wdelta_hbm, wdbuf)
    pltpu.sync_copy(bdelta_hbm, bdbuf)
    idx = ibuf[0:_L][0]
    iota = lax.iota(jnp.int32, _L)

    hvec = [hbuf[g * _L:(g + 1) * _L] for g in range(4)]
    bevec = [bentbuf[g * _L:(g + 1) * _L] for g in range(4)]

    pvec = []
    for jg in range(4):
        acc = jnp.zeros((_L,), jnp.float32)
        for k in range(_D):
            wrow = plsc.load_gather(wbuf, [k * _D + jg * _L + iota])
            acc = acc + hvec[k // _L][k % _L] * wrow
        pvec.append(acc)

    hbacc = jnp.zeros((_L,), jnp.float32)
    pn = jnp.zeros((_L,), jnp.float32)
    for g in range(4):
        hbacc = hbacc + hvec[g] * bevec[g]
        pn = pn + pvec[g] * pvec[g]
    hb = _vsum(hbacc)
    pnorm_s = _vsum(pn)
    shift = pnorm_s * _rsqrt16(jnp.full((_L,), pnorm_s, jnp.float32))[0] + hb

    g_owner = idx // _B
    owner_w = lax.rem(g_owner, _NW)
    owner_j = g_owner // _NW

    sebuf[...] = jnp.zeros((_L,), jnp.float32)
    tvbuf[...] = jnp.zeros((_L,), jnp.float32)

    n_j = (_NBLK - wid + _NW - 1) // _NW

    bufs = [buf0, buf1, buf2, buf3]
    lbufs = [lbuf0, lbuf1, lbuf2, lbuf3]

    def blk_start(j):
        return (wid + j * _NW) * _B

    def start_in(j, b):
        pltpu.async_copy(e_hbm.at[pl.ds(blk_start(j), _B), :], bufs[b],
                         in_sem)

    def wait_in(b):
        pltpu.make_async_copy(e_hbm.at[pl.ds(0, _B), :], bufs[b],
                              in_sem).wait()

    def start_out(j, b):
        pltpu.async_copy(bufs[b], eout_hbm.at[pl.ds(blk_start(j), _B), :],
                         out_sem)

    def wait_out(b):
        pltpu.make_async_copy(bufs[b], eout_hbm.at[pl.ds(0, _B), :],
                              out_sem).wait()

    def start_lg(j, b):
        pltpu.async_copy(lbufs[b], lg_hbm.at[pl.ds(blk_start(j), _B)], lg_sem)

    def wait_lg(b):
        pltpu.make_async_copy(lbufs[b], lg_hbm.at[pl.ds(0, _B)], lg_sem).wait()

    def compute(j, b):
        buf = bufs[b]
        lbuf = lbufs[b]

        def group_body(t, c2):
            rows = t * _L + iota
            acc = jnp.full((_L,), hb, jnp.float32)
            for d in range(_D):
                v = plsc.load_gather(buf, [rows, jnp.full((_L,), d, jnp.int32)])
                acc = acc + v * pvec[d // _L][d % _L]
            plsc.store_scatter(lbuf, [rows], acc)
            sebuf[...] = sebuf[...] + jnp.exp(acc - shift)
            return c2

        lax.fori_loop(0, _B // _L, group_body, 0)

        @pl.when((wid == owner_w) & (j == owner_j))
        def _update():
            g_own = wid + j * _NW
            lrow = idx - g_own * _B
            tg = lrow // _L
            lr = lrow - tg * _L
            lv = lbuf[pl.ds(tg * _L, _L)]
            tvbuf[...] = jnp.where(iota == lr, lv, 0.0)
            rowsplat = jnp.full((_L,), lrow, jnp.int32)
            e_g = [plsc.load_gather(buf, [rowsplat, q * _L + iota])
                   for q in range(4)]
            s = jnp.zeros((_L,), jnp.float32)
            for q in range(4):
                qacc = jnp.zeros((_L,), jnp.float32)
                for k in range(_D):
                    wcol = plsc.load_gather(
                        wdbuf, [(q * _L + iota) * _D + k])
                    qacc = qacc + e_g[k // _L][k % _L] * wcol
                qv = qacc + bdbuf[q * _L:(q + 1) * _L]
                s = s + hvec[q] * qv
            sv = jnp.full((_L,), _vsum(s), jnp.float32)
            delta = 1.0 / (1.0 + jnp.exp(-sv))
            nrm = jnp.zeros((_L,), jnp.float32)
            us = []
            for q in range(4):
                u = delta * e_g[q] + (1.0 - delta) * hvec[q]
                us.append(u)
                nrm = nrm + u * u
            rin = _rsqrt16(jnp.full((_L,), _vsum(nrm), jnp.float32))
            for q in range(4):
                plsc.store_scatter(buf, [rowsplat, q * _L + iota], us[q] * rin)

    for b in range(4):
        @pl.when(b < n_j)
        def _(b=b):
            start_in(b, b)

    n_outer = (_NBLK // _NW + 1 + 3) // 4 + 1

    def outer_body(t, carry):
        j0 = t * 4
        for b in range(4):
            @pl.when(j0 + b < n_j)
            def _(b=b):
                j = j0 + b
                wait_in(b)
                compute(j, b)
                start_out(j, b)
                start_lg(j, b)
        for b in range(4):
            @pl.when(j0 + 4 + b < n_j)
            def _(b=b):
                wait_out(b)
                wait_lg(b)
                start_in(j0 + 4 + b, b)
        return carry

    lax.fori_loop(0, n_outer, outer_body, 0)

    def drain_body(t, carry):
        wait_out(0)
        wait_lg(0)
        return carry

    lax.fori_loop(0, jnp.minimum(n_j, 4), drain_body, 0)

    se_tot = _vsum(sebuf[...])
    tv = _vsum(tvbuf[...])
    out16 = jnp.where(iota == 0, jnp.full((_L,), se_tot, jnp.float32),
                      jnp.where(iota == 1, jnp.full((_L,), tv, jnp.float32),
                                jnp.where(iota == 2,
                                          jnp.full((_L,), shift, jnp.float32),
                                          jnp.zeros((_L,), jnp.float32))))
    obuf[0:_L] = out16
    for g in range(1, 8):
        obuf[g * _L:(g + 1) * _L] = jnp.zeros((_L,), jnp.float32)
    pltpu.sync_copy(obuf, part_hbm.at[wid])


def _loss_body(part_ref, loss_ref):
    p = part_ref[...]
    lanei = jax.lax.broadcasted_iota(jnp.int32, (_NW, 128), 1)
    se = jnp.sum(jnp.where(lanei == 0, p, 0.0))
    tv = jnp.sum(jnp.where(lanei == 1, p, 0.0))
    shift = jnp.sum(jnp.where(lanei == 2, p, 0.0)) / _NW
    loss_ref[...] = jnp.full((1, 128), jnp.log(se) + shift - tv, jnp.float32)


def kernel(h, r, entity_idx, entity_embeddings, W_ent, b_ent, W_delta, b_delta):
    del r
    idx16 = jnp.broadcast_to(jnp.asarray(entity_idx, jnp.int32), (_L,))
    mesh = plsc.VectorSubcoreMesh(core_axis_name="c", subcore_axis_name="s")

    sc = functools.partial(
        pl.kernel,
        mesh=mesh,
        compiler_params=pltpu.CompilerParams(needs_layout_passes=False),
        out_type=[
            jax.ShapeDtypeStruct((_M, _D), jnp.float32),
            jax.ShapeDtypeStruct((_M,), jnp.float32),
            jax.ShapeDtypeStruct((_NW, 128), jnp.float32),
        ],
        scratch_types=[
            pltpu.VMEM((_B, _D), jnp.float32),
            pltpu.VMEM((_B, _D), jnp.float32),
            pltpu.VMEM((_B, _D), jnp.float32),
            pltpu.VMEM((_B, _D), jnp.float32),
            pltpu.VMEM((_B,), jnp.float32),
            pltpu.VMEM((_B,), jnp.float32),
            pltpu.VMEM((_B,), jnp.float32),
            pltpu.VMEM((_B,), jnp.float32),
            pltpu.VMEM((_D,), jnp.float32),
            pltpu.VMEM((_D * _D,), jnp.float32),
            pltpu.VMEM((_D,), jnp.float32),
            pltpu.VMEM((_D * _D,), jnp.float32),
            pltpu.VMEM((_D,), jnp.float32),
            pltpu.VMEM((_L,), jnp.int32),
            pltpu.VMEM((_L,), jnp.float32),
            pltpu.VMEM((_L,), jnp.float32),
            pltpu.VMEM((128,), jnp.float32),
            pltpu.SemaphoreType.DMA,
            pltpu.SemaphoreType.DMA,
            pltpu.SemaphoreType.DMA,
        ],
    )(_sc_body)

    eout, lg, part = sc(idx16, entity_embeddings, h,
                        W_ent.reshape(_D * _D), b_ent,
                        W_delta.reshape(_D * _D), b_delta)

    loss_v = pl.pallas_call(
        _loss_body,
        in_specs=[pl.BlockSpec((_NW, 128), lambda: (0, 0))],
        out_specs=pl.BlockSpec((1, 128), lambda: (0, 0)),
        out_shape=jax.ShapeDtypeStruct((1, 128), jnp.float32),
    )(part)

    return lg, loss_v[0, 0], eout


# speedup vs baseline: 1.0547x; 1.0343x over previous
import functools

import jax
import jax.numpy as jnp
from jax import lax
from jax.experimental import pallas as pl
from jax.experimental.pallas import tpu as pltpu
from jax.experimental.pallas import tpu_sc as plsc

_M = 1000000
_D = 64
_B = 160
_NBLK = _M // _B
_NW = 32
_L = 16


def _rsqrt16(x):
    i = plsc.bitcast(x, jnp.int32)
    i = jnp.int32(0x5F3759DF) - lax.shift_right_logical(i, 1)
    y = plsc.bitcast(i, jnp.float32)
    for _ in range(3):
        y = y * (1.5 - 0.5 * x * y * y)
    return y


def _vsum(v):
    return lax.reduce_sum_p.bind(v, axes=(0,))


def _sc_body(idx_hbm, e_hbm, h_hbm, went_hbm, bent_hbm, wdelta_hbm,
             bdelta_hbm, eout_hbm, lg_hbm, part_hbm,
             buf0, buf1, buf2, buf3, lbuf0, lbuf1, lbuf2, lbuf3,
             hbuf, wbuf, bentbuf, wdbuf, bdbuf, ibuf,
             sebuf, tvbuf, obuf, in_sem, out_sem, lg_sem):
    wid = lax.axis_index("s") * 2 + lax.axis_index("c")

    pltpu.sync_copy(idx_hbm, ibuf)
    pltpu.sync_copy(h_hbm, hbuf)
    pltpu.sync_copy(went_hbm, wbuf)
    pltpu.sync_copy(bent_hbm, bentbuf)
    pltpu.sync_copy(wdelta_hbm, wdbuf)
    pltpu.sync_copy(bdelta_hbm, bdbuf)
    idx = ibuf[0:_L][0]
    iota = lax.iota(jnp.int32, _L)

    hvec = [hbuf[g * _L:(g + 1) * _L] for g in range(4)]
    bevec = [bentbuf[g * _L:(g + 1) * _L] for g in range(4)]

    pvec = []
    for jg in range(4):
        acc = jnp.zeros((_L,), jnp.float32)
        for k in range(_D):
            wrow = plsc.load_gather(wbuf, [k * _D + jg * _L + iota])
            acc = acc + hvec[k // _L][k % _L] * wrow
        pvec.append(acc)

    hbacc = jnp.zeros((_L,), jnp.float32)
    pn = jnp.zeros((_L,), jnp.float32)
    for g in range(4):
        hbacc = hbacc + hvec[g] * bevec[g]
        pn = pn + pvec[g] * pvec[g]
    hb = _vsum(hbacc)
    pnorm_s = _vsum(pn)
    shift = pnorm_s * _rsqrt16(jnp.full((_L,), pnorm_s, jnp.float32))[0] + hb

    g_owner = idx // _B
    owner_w = lax.rem(g_owner, _NW)
    owner_j = g_owner // _NW

    sebuf[...] = jnp.zeros((_L,), jnp.float32)
    tvbuf[...] = jnp.zeros((_L,), jnp.float32)

    n_j = (_NBLK - wid + _NW - 1) // _NW

    bufs = [buf0, buf1, buf2, buf3]
    lbufs = [lbuf0, lbuf1, lbuf2, lbuf3]

    def blk_start(j):
        return (wid + j * _NW) * _B

    def start_in(j, b):
        pltpu.async_copy(e_hbm.at[pl.ds(blk_start(j), _B), :], bufs[b],
                         in_sem)

    def wait_in(b):
        pltpu.make_async_copy(e_hbm.at[pl.ds(0, _B), :], bufs[b],
                              in_sem).wait()

    def start_out(j, b):
        pltpu.async_copy(bufs[b], eout_hbm.at[pl.ds(blk_start(j), _B), :],
                         out_sem)

    def wait_out(b):
        pltpu.make_async_copy(bufs[b], eout_hbm.at[pl.ds(0, _B), :],
                              out_sem).wait()

    def start_lg(j, b):
        pltpu.async_copy(lbufs[b], lg_hbm.at[pl.ds(blk_start(j), _B)], lg_sem)

    def wait_lg(b):
        pltpu.make_async_copy(lbufs[b], lg_hbm.at[pl.ds(0, _B)], lg_sem).wait()

    def compute(j, b):
        buf = bufs[b]
        lbuf = lbufs[b]

        def group_body(t, c2):
            rows = t * _L + iota
            accs = [jnp.full((_L,), hb, jnp.float32)] + [
                jnp.zeros((_L,), jnp.float32) for _ in range(3)]
            for d in range(_D):
                v = plsc.load_gather(buf, [rows, jnp.full((_L,), d, jnp.int32)])
                accs[d % 4] = accs[d % 4] + v * pvec[d // _L][d % _L]
            acc = (accs[0] + accs[1]) + (accs[2] + accs[3])
            plsc.store_scatter(lbuf, [rows], acc)
            sebuf[...] = sebuf[...] + jnp.exp(acc - shift)
            return c2

        lax.fori_loop(0, _B // _L, group_body, 0)

        @pl.when((wid == owner_w) & (j == owner_j))
        def _update():
            g_own = wid + j * _NW
            lrow = idx - g_own * _B
            tg = lrow // _L
            lr = lrow - tg * _L
            lv = lbuf[pl.ds(tg * _L, _L)]
            tvbuf[...] = jnp.where(iota == lr, lv, 0.0)
            rowsplat = jnp.full((_L,), lrow, jnp.int32)
            e_g = [plsc.load_gather(buf, [rowsplat, q * _L + iota])
                   for q in range(4)]
            s = jnp.zeros((_L,), jnp.float32)
            for q in range(4):
                qacc = jnp.zeros((_L,), jnp.float32)
                for k in range(_D):
                    wcol = plsc.load_gather(
                        wdbuf, [(q * _L + iota) * _D + k])
                    qacc = qacc + e_g[k // _L][k % _L] * wcol
                qv = qacc + bdbuf[q * _L:(q + 1) * _L]
                s = s + hvec[q] * qv
            sv = jnp.full((_L,), _vsum(s), jnp.float32)
            delta = 1.0 / (1.0 + jnp.exp(-sv))
            nrm = jnp.zeros((_L,), jnp.float32)
            us = []
            for q in range(4):
                u = delta * e_g[q] + (1.0 - delta) * hvec[q]
                us.append(u)
                nrm = nrm + u * u
            rin = _rsqrt16(jnp.full((_L,), _vsum(nrm), jnp.float32))
            for q in range(4):
                plsc.store_scatter(buf, [rowsplat, q * _L + iota], us[q] * rin)

    for b in range(4):
        @pl.when(b < n_j)
        def _(b=b):
            start_in(b, b)

    n_outer = (_NBLK // _NW + 1 + 3) // 4 + 1

    def outer_body(t, carry):
        j0 = t * 4
        for b in range(4):
            @pl.when(j0 + b < n_j)
            def _(b=b):
                j = j0 + b
                wait_in(b)
                compute(j, b)
                start_out(j, b)
                start_lg(j, b)
        for b in range(4):
            @pl.when(j0 + 4 + b < n_j)
            def _(b=b):
                wait_out(b)
                wait_lg(b)
                start_in(j0 + 4 + b, b)
        return carry

    lax.fori_loop(0, n_outer, outer_body, 0)

    def drain_body(t, carry):
        wait_out(0)
        wait_lg(0)
        return carry

    lax.fori_loop(0, jnp.minimum(n_j, 4), drain_body, 0)

    se_tot = _vsum(sebuf[...])
    tv = _vsum(tvbuf[...])
    out16 = jnp.where(iota == 0, jnp.full((_L,), se_tot, jnp.float32),
                      jnp.where(iota == 1, jnp.full((_L,), tv, jnp.float32),
                                jnp.where(iota == 2,
                                          jnp.full((_L,), shift, jnp.float32),
                                          jnp.zeros((_L,), jnp.float32))))
    obuf[0:_L] = out16
    for g in range(1, 8):
        obuf[g * _L:(g + 1) * _L] = jnp.zeros((_L,), jnp.float32)
    pltpu.sync_copy(obuf, part_hbm.at[wid])


def _loss_body(part_ref, loss_ref):
    p = part_ref[...]
    lanei = jax.lax.broadcasted_iota(jnp.int32, (_NW, 128), 1)
    se = jnp.sum(jnp.where(lanei == 0, p, 0.0))
    tv = jnp.sum(jnp.where(lanei == 1, p, 0.0))
    shift = jnp.sum(jnp.where(lanei == 2, p, 0.0)) / _NW
    loss_ref[...] = jnp.full((1, 128), jnp.log(se) + shift - tv, jnp.float32)


def kernel(h, r, entity_idx, entity_embeddings, W_ent, b_ent, W_delta, b_delta):
    del r
    idx16 = jnp.broadcast_to(jnp.asarray(entity_idx, jnp.int32), (_L,))
    mesh = plsc.VectorSubcoreMesh(core_axis_name="c", subcore_axis_name="s")

    sc = functools.partial(
        pl.kernel,
        mesh=mesh,
        compiler_params=pltpu.CompilerParams(needs_layout_passes=False),
        out_type=[
            jax.ShapeDtypeStruct((_M, _D), jnp.float32),
            jax.ShapeDtypeStruct((_M,), jnp.float32),
            jax.ShapeDtypeStruct((_NW, 128), jnp.float32),
        ],
        scratch_types=[
            pltpu.VMEM((_B, _D), jnp.float32),
            pltpu.VMEM((_B, _D), jnp.float32),
            pltpu.VMEM((_B, _D), jnp.float32),
            pltpu.VMEM((_B, _D), jnp.float32),
            pltpu.VMEM((_B,), jnp.float32),
            pltpu.VMEM((_B,), jnp.float32),
            pltpu.VMEM((_B,), jnp.float32),
            pltpu.VMEM((_B,), jnp.float32),
            pltpu.VMEM((_D,), jnp.float32),
            pltpu.VMEM((_D * _D,), jnp.float32),
            pltpu.VMEM((_D,), jnp.float32),
            pltpu.VMEM((_D * _D,), jnp.float32),
            pltpu.VMEM((_D,), jnp.float32),
            pltpu.VMEM((_L,), jnp.int32),
            pltpu.VMEM((_L,), jnp.float32),
            pltpu.VMEM((_L,), jnp.float32),
            pltpu.VMEM((128,), jnp.float32),
            pltpu.SemaphoreType.DMA,
            pltpu.SemaphoreType.DMA,
            pltpu.SemaphoreType.DMA,
        ],
    )(_sc_body)

    eout, lg, part = sc(idx16, entity_embeddings, h,
                        W_ent.reshape(_D * _D), b_ent,
                        W_delta.reshape(_D * _D), b_delta)

    loss_v = pl.pallas_call(
        _loss_body,
        in_specs=[pl.BlockSpec((_NW, 128), lambda: (0, 0))],
        out_specs=pl.BlockSpec((1, 128), lambda: (0, 0)),
        out_shape=jax.ShapeDtypeStruct((1, 128), jnp.float32),
    )(part)

    return lg, loss_v[0, 0], eout
